# Initial kernel scaffold; baseline (speedup 1.0000x reference)
import jax
import jax.numpy as jnp
from jax.experimental import pallas as pl


def kernel(joints, joints_feature, cls_logits, mask, params, codebook):
    raise NotImplementedError("write your pallas kernel here")


# speedup vs baseline: 1.6328x; 1.6328x over previous
import functools

import jax
import jax.numpy as jnp
from jax.experimental import pallas as pl
from jax.experimental.pallas import tpu as pltpu

_BS = 256
_NJ = 17
_HID = 512
_TN = 34
_TD = 64
_K = 8192
_DH = 32

_EBB = 64
_VROWS = 544
_VCHUNK = 2048


def _lnk(x, g, b):
    m = jnp.mean(x, axis=-1, keepdims=True)
    v = jnp.mean((x - m) ** 2, axis=-1, keepdims=True)
    return (x - m) / jnp.sqrt(v + 1e-5) * g + b


def _gelu(x):
    return x * 0.5 * (1.0 + jax.lax.erf(x * (2.0 ** -0.5)))


def _tok_mix(x, bb, nj, hid, w1, b1, w2, b2):
    xt = jnp.swapaxes(x.reshape(bb, nj, hid), 1, 2).reshape(bb * hid, nj)
    h = _gelu(jnp.dot(xt, w1) + b1)
    yt = jnp.dot(h, w2) + b2
    return jnp.swapaxes(yt.reshape(bb, hid, nj), 1, 2).reshape(bb * nj, hid)


def _mixer_block(x, bb, nj, hid, ws):
    (g1, b1, tw1, tb1, tw2, tb2, g2, b2, cw1, cb1, cw2, cb2) = ws
    y = _tok_mix(_lnk(x, g1, b1), bb, nj, hid, tw1, tb1, tw2, tb2)
    xy = x + y
    z = _lnk(xy, g2, b2)
    z = jnp.dot(_gelu(jnp.dot(z, cw1) + cb1), cw2) + cb2
    return xy + z


def _enc_body(bb, j_ref, m_ref, *refs):
    out_ref = refs[-1]
    ws = [r[...] for r in refs[:-1]]
    sw, sb, inv = ws[0:3]
    layers = [ws[3 + 12 * l: 3 + 12 * (l + 1)] for l in range(4)]
    eg, eb, tw, tb, fw, fb = ws[3 + 48: 3 + 48 + 6]

    j = j_ref[...]
    x = j[:, 0:1] * sw[0:1, :] + j[:, 1:2] * sw[1:2, :] + j[:, 2:3] * sw[2:3, :] + sb
    m = m_ref[...]
    x = x * m + inv * (1.0 - m)
    for ls in layers:
        x = _mixer_block(x, bb, _NJ, _HID, ls)
    x = _lnk(x, eg, eb)
    xt = jnp.swapaxes(x.reshape(bb, _NJ, _HID), 1, 2).reshape(bb * _HID, _NJ)
    t = jnp.dot(xt, tw) + tb
    t = jnp.swapaxes(t.reshape(bb, _HID, _TN), 1, 2).reshape(bb * _TN, _HID)
    out_ref[...] = jnp.dot(t, fw) + fb


def _vq_body(nchunks, rows, ef_ref, cbt_ref, cb_ref, idx_ref, part_ref, loss_ref):
    i = pl.program_id(0)
    ef = ef_ref[...]
    efn = jnp.sum(ef * ef, axis=1, keepdims=True)
    cw = _K // nchunks
    best_d = jnp.full((rows, 1), jnp.inf, jnp.float32)
    best_i = jnp.full((rows, 1), 0, jnp.int32)
    for c in range(nchunks):
        cbt_c = cbt_ref[:, c * cw:(c + 1) * cw]
        cn = jnp.sum(cbt_c * cbt_c, axis=0, keepdims=True)
        d = efn + cn - 2.0 * jnp.dot(ef, cbt_c)
        mc = jnp.min(d, axis=1, keepdims=True)
        cols = jax.lax.broadcasted_iota(jnp.int32, (rows, cw), 1)
        ac = jnp.min(jnp.where(d == mc, cols, cw), axis=1, keepdims=True) + c * cw
        upd = mc < best_d
        best_d = jnp.where(upd, mc, best_d)
        best_i = jnp.where(upd, ac, best_i)
    part = jnp.zeros((rows, _TD), jnp.float32)
    for c in range(nchunks):
        cols = jax.lax.broadcasted_iota(jnp.int32, (rows, cw), 1) + c * cw
        oh = (cols == best_i).astype(jnp.float32)
        part = part + jnp.dot(oh, cb_ref[c * cw:(c + 1) * cw, :])
    idx_ref[...] = best_i
    part_ref[...] = part

    @pl.when(i == 0)
    def _():
        loss_ref[...] = jnp.zeros((1, 1), jnp.float32)

    loss_ref[...] += jnp.sum((part - ef) ** 2).reshape(1, 1)

    @pl.when(i == pl.num_programs(0) - 1)
    def _():
        loss_ref[...] = loss_ref[...] / float(_BS * _TN * _TD)


def _dec_body(bb, p_ref, *refs):
    out_ref = refs[-1]
    ws = [r[...] for r in refs[:-1]]
    tokw, tokb, dsw, dsb = ws[0:4]
    mix = ws[4:16]
    dg, db, rw, rb = ws[16:20]

    p = p_ref[...]
    pt = jnp.swapaxes(p, 1, 2).reshape(bb * _TD, _TN)
    q = jnp.dot(pt, tokw) + tokb
    q = jnp.swapaxes(q.reshape(bb, _TD, _NJ), 1, 2).reshape(bb * _NJ, _TD)
    x = jnp.dot(q, dsw) + dsb
    x = _mixer_block(x, bb, _NJ, _DH, mix)
    x = _lnk(x, dg, db)
    rec = jnp.dot(x, rw) + rb
    out_ref[...] = rec.reshape(bb, _NJ, 3)


def _full_spec(ndim):
    if ndim == 2:
        return lambda i: (0, 0)
    return lambda i: (0, 0, 0)


def kernel(joints, joints_feature, cls_logits, mask, params, codebook):
    p = params
    maskf = mask.astype(jnp.float32)

    def v2(x):
        return x.reshape(1, -1)

    def mixer_ops(lp):
        return [v2(lp['ln1_g']), v2(lp['ln1_b']), lp['tw1'], v2(lp['tb1']),
                lp['tw2'], v2(lp['tb2']), v2(lp['ln2_g']), v2(lp['ln2_b']),
                lp['cw1'], v2(lp['cb1']), lp['cw2'], v2(lp['cb2'])]

    enc_ops = [p['start_w'], v2(p['start_b']), p['invisible_token'].reshape(1, _HID)]
    for lp in p['enc_layers']:
        enc_ops += mixer_ops(lp)
    enc_ops += [v2(p['enc_ln_g']), v2(p['enc_ln_b']), p['token_mlp_w'],
                v2(p['token_mlp_b']), p['feat_w'], v2(p['feat_b'])]

    ef = pl.pallas_call(
        functools.partial(_enc_body, _EBB),
        grid=(_BS // _EBB,),
        in_specs=[pl.BlockSpec((_EBB * _NJ, 3), lambda i: (i, 0)),
                  pl.BlockSpec((_EBB * _NJ, 1), lambda i: (i, 0))]
                 + [pl.BlockSpec(op.shape, _full_spec(op.ndim)) for op in enc_ops],
        out_specs=pl.BlockSpec((_EBB * _TN, _TD), lambda i: (i, 0)),
        out_shape=jax.ShapeDtypeStruct((_BS * _TN, _TD), jnp.float32),
    )(joints.reshape(_BS * _NJ, 3), maskf.reshape(_BS * _NJ, 1), *enc_ops)

    nblk = (_BS * _TN) // _VROWS
    idx2, part, loss = pl.pallas_call(
        functools.partial(_vq_body, _K // _VCHUNK, _VROWS),
        grid=(nblk,),
        in_specs=[pl.BlockSpec((_VROWS, _TD), lambda i: (i, 0)),
                  pl.BlockSpec((_TD, _K), lambda i: (0, 0)),
                  pl.BlockSpec((_K, _TD), lambda i: (0, 0))],
        out_specs=[pl.BlockSpec((_VROWS, 1), lambda i: (i, 0)),
                   pl.BlockSpec((_VROWS, _TD), lambda i: (i, 0)),
                   pl.BlockSpec((1, 1), lambda i: (0, 0))],
        out_shape=[jax.ShapeDtypeStruct((_BS * _TN, 1), jnp.int32),
                   jax.ShapeDtypeStruct((_BS * _TN, _TD), jnp.float32),
                   jax.ShapeDtypeStruct((1, 1), jnp.float32)],
        compiler_params=pltpu.CompilerParams(dimension_semantics=("arbitrary",)),
    )(ef, codebook.T, codebook)

    dec_ops = [p['dec_tok_w'], v2(p['dec_tok_b']), p['dec_start_w'], v2(p['dec_start_b'])]
    dec_ops += mixer_ops(p['dec_layers'][0])
    dec_ops += [v2(p['dec_ln_g']), v2(p['dec_ln_b']), p['rec_w'], v2(p['rec_b'])]

    rec = pl.pallas_call(
        functools.partial(_dec_body, _BS),
        grid=(1,),
        in_specs=[pl.BlockSpec((_BS, _TN, _TD), lambda i: (0, 0, 0))]
                 + [pl.BlockSpec(op.shape, _full_spec(op.ndim)) for op in dec_ops],
        out_specs=pl.BlockSpec((_BS, _NJ, 3), lambda i: (0, 0, 0)),
        out_shape=jax.ShapeDtypeStruct((_BS, _NJ, 3), jnp.float32),
    )(part.reshape(_BS, _TN, _TD), *dec_ops)

    return rec, idx2.reshape(-1), loss[0, 0]
